# Initial kernel scaffold; baseline (speedup 1.0000x reference)
import jax
import jax.numpy as jnp
from jax.experimental import pallas as pl


def kernel(batch_token_ids, token_embedding):
    raise NotImplementedError("write your pallas kernel here")


# speedup vs baseline: 5.1852x; 5.1852x over previous
import functools

import jax
import jax.numpy as jnp
from jax import lax
from jax.experimental import pallas as pl
from jax.experimental.pallas import tpu as pltpu
from jax.experimental.pallas import tpu_sc as plsc

HIDDEN = 128
NC = 2
NS = 16
NW = NC * NS

LANE = 128


def _make_gather(n_idx_rows):
    rows_per_w = n_idx_rows // NW
    mesh = plsc.VectorSubcoreMesh(core_axis_name="c", subcore_axis_name="s")

    @functools.partial(
        pl.kernel,
        mesh=mesh,
        out_type=jax.ShapeDtypeStruct((n_idx_rows * LANE, HIDDEN), jnp.float32),
        scratch_types=[
            pltpu.VMEM((LANE,), jnp.int32),
            pltpu.VMEM((LANE, HIDDEN), jnp.float32),
            pltpu.SemaphoreType.DMA,
        ],
    )
    def gather_kernel(idx_hbm, table_hbm, out_hbm, idx_v, rows_v, sem):
        wid = lax.axis_index("s") * NC + lax.axis_index("c")
        w_row0 = wid * rows_per_w

        def step(g, carry):
            row0 = w_row0 + g
            pltpu.sync_copy(idx_hbm.at[row0], idx_v)
            pltpu.async_copy(table_hbm.at[idx_v], rows_v, sem).wait()
            pltpu.sync_copy(rows_v, out_hbm.at[pl.ds(row0 * LANE, LANE)])
            return carry

        lax.fori_loop(0, rows_per_w, step, 0)

    return gather_kernel


def kernel(batch_token_ids, token_embedding):
    b, s = batch_token_ids.shape
    n = b * s
    idx2d = batch_token_ids.reshape(n // LANE, LANE).astype(jnp.int32)
    out = _make_gather(n // LANE)(idx2d, token_embedding)
    return out.reshape(b, s, HIDDEN)


# speedup vs baseline: 5.8096x; 1.1204x over previous
import functools

import jax
import jax.numpy as jnp
from jax import lax
from jax.experimental import pallas as pl
from jax.experimental.pallas import tpu as pltpu
from jax.experimental.pallas import tpu_sc as plsc

HIDDEN = 128
NC = 2
NS = 16
NW = NC * NS

LANE = 128


def _make_gather(n_idx_rows):
    rows_per_w = n_idx_rows // NW
    mesh = plsc.VectorSubcoreMesh(core_axis_name="c", subcore_axis_name="s")

    @functools.partial(
        pl.kernel,
        mesh=mesh,
        out_type=jax.ShapeDtypeStruct((n_idx_rows * LANE, HIDDEN), jnp.float32),
        scratch_types=[
            pltpu.VMEM((LANE,), jnp.int32),
            pltpu.VMEM((LANE, HIDDEN), jnp.float32),
            pltpu.VMEM((LANE, HIDDEN), jnp.float32),
            pltpu.SemaphoreType.DMA,
            pltpu.SemaphoreType.DMA,
            pltpu.SemaphoreType.DMA,
        ],
    )
    def gather_kernel(idx_hbm, table_hbm, out_hbm, idx_v, rows_v0, rows_v1,
                      gsem, osem0, osem1):
        wid = lax.axis_index("s") * NC + lax.axis_index("c")
        w_row0 = wid * rows_per_w
        bufs = ((rows_v0, osem0), (rows_v1, osem1))

        def drain_outs():
            for rv, osem in bufs:
                pltpu.make_async_copy(
                    rv, out_hbm.at[pl.ds(0, LANE)], osem
                ).wait()

        def pair(p, carry):
            pl.when(p >= 1)(drain_outs)
            for b, (rv, osem) in enumerate(bufs):
                row0 = w_row0 + 2 * p + b
                pltpu.sync_copy(idx_hbm.at[row0], idx_v)
                pltpu.async_copy(table_hbm.at[idx_v], rv, gsem).wait()
                pltpu.async_copy(rv, out_hbm.at[pl.ds(row0 * LANE, LANE)], osem)
            return carry

        lax.fori_loop(0, rows_per_w // 2, pair, 0)
        drain_outs()

    return gather_kernel


def kernel(batch_token_ids, token_embedding):
    b, s = batch_token_ids.shape
    n = b * s
    idx2d = batch_token_ids.reshape(n // LANE, LANE).astype(jnp.int32)
    out = _make_gather(n // LANE)(idx2d, token_embedding)
    return out.reshape(b, s, HIDDEN)


# speedup vs baseline: 9.2222x; 1.5874x over previous
import functools

import jax
import jax.numpy as jnp
from jax import lax
from jax.experimental import pallas as pl
from jax.experimental.pallas import tpu as pltpu
from jax.experimental.pallas import tpu_sc as plsc

HIDDEN = 128
NC = 2
NS = 16
NW = NC * NS

LANE = 128


def _make_gather(n_idx_rows):
    rows_per_w = n_idx_rows // NW
    mesh = plsc.VectorSubcoreMesh(core_axis_name="c", subcore_axis_name="s")

    @functools.partial(
        pl.kernel,
        mesh=mesh,
        out_type=jax.ShapeDtypeStruct((n_idx_rows * LANE, HIDDEN), jnp.float32),
        scratch_types=(
            [pltpu.VMEM((LANE,), jnp.int32)] * 4
            + [pltpu.VMEM((LANE, HIDDEN), jnp.float32)] * 4
            + [pltpu.SemaphoreType.DMA] * 8
        ),
    )
    def gather_kernel(idx_hbm, table_hbm, out_hbm,
                      iv0, iv1, iv2, iv3, rv0, rv1, rv2, rv3,
                      gs0, gs1, gs2, gs3, os0, os1, os2, os3):
        wid = lax.axis_index("s") * NC + lax.axis_index("c")
        w_row0 = wid * rows_per_w
        n_quads = rows_per_w // 4
        IV = (iv0, iv1, iv2, iv3)
        RV = (rv0, rv1, rv2, rv3)
        GS = (gs0, gs1, gs2, gs3)
        OS = (os0, os1, os2, os3)

        def drain_out(b):
            pltpu.make_async_copy(RV[b], out_hbm.at[pl.ds(0, LANE)],
                                  OS[b]).wait()

        def fire_gather(b, row0):
            pltpu.sync_copy(idx_hbm.at[row0], IV[b])
            pltpu.async_copy(table_hbm.at[IV[b]], RV[b], GS[b])

        fire_gather(0, w_row0)
        fire_gather(1, w_row0 + 1)

        def quad(q, carry):
            for b in range(4):
                g = 4 * q + b
                bf = (b + 2) % 4
                if b < 2:
                    pl.when(q >= 1)(lambda bf=bf: drain_out(bf))
                    fire_gather(bf, w_row0 + g + 2)
                else:
                    drain_out(bf)
                    pl.when(q < n_quads - 1)(
                        lambda bf=bf, g=g: fire_gather(bf, w_row0 + g + 2))
                pltpu.make_async_copy(table_hbm.at[IV[b]], RV[b],
                                      GS[b]).wait()
                pltpu.async_copy(
                    RV[b], out_hbm.at[pl.ds((w_row0 + g) * LANE, LANE)],
                    OS[b])
            return carry

        lax.fori_loop(0, n_quads, quad, 0)
        drain_out(2)
        drain_out(3)

    return gather_kernel


def kernel(batch_token_ids, token_embedding):
    b, s = batch_token_ids.shape
    n = b * s
    idx2d = batch_token_ids.reshape(n // LANE, LANE).astype(jnp.int32)
    out = _make_gather(n // LANE)(idx2d, token_embedding)
    return out.reshape(b, s, HIDDEN)


# speedup vs baseline: 9.2410x; 1.0020x over previous
import functools

import jax
import jax.numpy as jnp
from jax import lax
from jax.experimental import pallas as pl
from jax.experimental.pallas import tpu as pltpu
from jax.experimental.pallas import tpu_sc as plsc

HIDDEN = 128
NC = 2
NS = 16
NW = NC * NS

LANE = 128
K = 5
J = 3


def _make_gather(n_idx_rows):
    rows_per_w = n_idx_rows // NW
    n_groups = rows_per_w // K
    mesh = plsc.VectorSubcoreMesh(core_axis_name="c", subcore_axis_name="s")

    @functools.partial(
        pl.kernel,
        mesh=mesh,
        out_type=jax.ShapeDtypeStruct((n_idx_rows * LANE, HIDDEN), jnp.float32),
        scratch_types=(
            [pltpu.VMEM((LANE,), jnp.int32)] * K
            + [pltpu.VMEM((LANE, HIDDEN), jnp.float32)] * K
            + [pltpu.SemaphoreType.DMA] * (2 * K)
        ),
    )
    def gather_kernel(idx_hbm, table_hbm, out_hbm, *refs):
        wid = lax.axis_index("s") * NC + lax.axis_index("c")
        w_row0 = wid * rows_per_w
        IV = refs[0:K]
        RV = refs[K:2 * K]
        GS = refs[2 * K:3 * K]
        OS = refs[3 * K:4 * K]

        def drain_out(b):
            pltpu.make_async_copy(RV[b], out_hbm.at[pl.ds(0, LANE)],
                                  OS[b]).wait()

        def fire_gather(b, row0):
            pltpu.sync_copy(idx_hbm.at[row0], IV[b])
            pltpu.async_copy(table_hbm.at[IV[b]], RV[b], GS[b])

        for b in range(J):
            fire_gather(b, w_row0 + b)

        def group(q, carry):
            for b in range(K):
                g = K * q + b
                bf = (b + J) % K
                if b < K - J:
                    pl.when(q >= 1)(lambda bf=bf: drain_out(bf))
                    fire_gather(bf, w_row0 + g + J)
                else:
                    drain_out(bf)
                    pl.when(q < n_groups - 1)(
                        lambda bf=bf, g=g: fire_gather(bf, w_row0 + g + J))
                pltpu.make_async_copy(table_hbm.at[IV[b]], RV[b],
                                      GS[b]).wait()
                pltpu.async_copy(
                    RV[b], out_hbm.at[pl.ds((w_row0 + g) * LANE, LANE)],
                    OS[b])
            return carry

        lax.fori_loop(0, n_groups, group, 0)
        for t in range(rows_per_w - (K - J), rows_per_w):
            drain_out(t % K)

    return gather_kernel


def kernel(batch_token_ids, token_embedding):
    b, s = batch_token_ids.shape
    n = b * s
    idx2d = batch_token_ids.reshape(n // LANE, LANE).astype(jnp.int32)
    out = _make_gather(n // LANE)(idx2d, token_embedding)
    return out.reshape(b, s, HIDDEN)
